# baseline (speedup 1.0000x reference)
import functools

import jax
import jax.numpy as jnp
from jax import lax
from jax.experimental import pallas as pl
from jax.experimental.pallas import tpu as pltpu
from jax.experimental.pallas import tpu_sc as plsc

N = 10000
E = 160000
D = 256
H = 128
EMBED = 256
C = 128
B = 1024

NC = 2
NS = 16
NW = NC * NS
L = 16

EPT = E // NW
EPT16 = 5008
NVEC = EPT16 // L
K = 112
CSIZE = EPT16 + K
SLOT_N = 10112
TRASH = B
ZROWS = 72
ACC_ROWS = NS * ZROWS
BPT = B // NS


def _sc_aggregate(x, src, dst, nodes, zacc, trash_tab):
    mesh = plsc.VectorSubcoreMesh(core_axis_name="c", subcore_axis_name="s",
                                  num_cores=NC, num_subcores=NS)

    @functools.partial(
        pl.kernel,
        out_type=(
            jax.ShapeDtypeStruct((NC * B, H), jnp.float32),
            jax.ShapeDtypeStruct((NC * B, H), jnp.float32),
            jax.ShapeDtypeStruct((NC * B,), jnp.float32),
            jax.ShapeDtypeStruct((B, D), jnp.float32),
        ),
        mesh=mesh,
        compiler_params=pltpu.CompilerParams(needs_layout_passes=False),
        scratch_types=[
            pltpu.VMEM((SLOT_N,), jnp.int32),
            pltpu.VMEM((B,), jnp.int32),
            pltpu.VMEM((EPT16,), jnp.int32),
            pltpu.VMEM((EPT16,), jnp.int32),
            pltpu.VMEM((CSIZE,), jnp.int32),
            pltpu.VMEM((CSIZE,), jnp.int32),
            pltpu.VMEM((K, H), jnp.float32),
            pltpu.VMEM((K, H), jnp.float32),
            pltpu.VMEM((K, H), jnp.float32),
            pltpu.VMEM((K, H), jnp.float32),
            pltpu.VMEM((BPT,), jnp.int32),
            pltpu.VMEM((BPT,), jnp.int32),
            pltpu.VMEM((BPT, H), jnp.float32),
            pltpu.VMEM((BPT, H), jnp.float32),
            pltpu.VMEM((K,), jnp.float32),
            pltpu.VMEM((K,), jnp.float32),
            pltpu.VMEM((80,), jnp.float32),
            pltpu.VMEM_SHARED((ACC_ROWS, H), jnp.float32),
            pltpu.VMEM_SHARED((ACC_ROWS, H), jnp.float32),
            pltpu.VMEM_SHARED((ACC_ROWS,), jnp.float32),
            pltpu.SemaphoreType.DMA,
            pltpu.SemaphoreType.DMA,
            pltpu.SemaphoreType.DMA,
            pltpu.SemaphoreType.DMA,
            pltpu.SemaphoreType.DMA,
            pltpu.SemaphoreType.DMA,
            pltpu.SemaphoreType.DMA,
            pltpu.SemaphoreType.DMA,
            pltpu.SemaphoreType.DMA,
            pltpu.SemaphoreType.DMA,
        ],
    )
    def sc(x_hbm, src_hbm, dst_hbm, nodes_hbm, zacc_hbm, trash_hbm,
           nlo_hbm, nhi_hbm, dpart_hbm, xb_hbm,
           slot_v, nodes_v, dstbuf, srcbuf, csrc, cslot,
           glo0, ghi0, glo1, ghi1,
           eidx, oidx, xlo, xhi, onesbuf, degbuf, zbuf,
           acc_lo, acc_hi, deg_sh,
           gsem0, gsem1, slsem, shsem, sdsem, xsem1, xsem2, xsem3,
           xgsem1, xgsem2):
        cid = lax.axis_index("c")
        sid = lax.axis_index("s")
        gwid = cid * NS + sid
        ebase = gwid * EPT

        iota16 = lax.iota(jnp.int32, L)
        glos = (glo0, glo1)
        ghis = (ghi0, ghi1)
        gsems = (gsem0, gsem1)

        c_zl = pltpu.async_copy(zacc_hbm, acc_lo.at[pl.ds(sid * ZROWS, ZROWS)],
                                xsem1)
        c_zh = pltpu.async_copy(zacc_hbm, acc_hi.at[pl.ds(sid * ZROWS, ZROWS)],
                                xsem2)
        c_st = pltpu.async_copy(trash_hbm, slot_v, xsem3)
        c_nd = pltpu.async_copy(nodes_hbm, nodes_v, gsem0)
        pad_dst = jnp.full((L,), N, jnp.int32)
        dstbuf[pl.ds(EPT16 - L, L)] = pad_dst
        c_ed = pltpu.async_copy(dst_hbm.at[pl.ds(ebase, EPT)],
                                dstbuf.at[pl.ds(0, EPT)], gsem1)
        c_es = pltpu.async_copy(src_hbm.at[pl.ds(ebase, EPT)],
                                srcbuf.at[pl.ds(0, EPT)], slsem)

        zero16 = jnp.zeros((L,), jnp.float32)
        for i in range(80 // L):
            zbuf[pl.ds(i * L, L)] = zero16
        one16 = jnp.ones((L,), jnp.float32)
        for i in range(K // L):
            onesbuf[pl.ds(i * L, L)] = one16
        c_zd = pltpu.async_copy(zbuf.at[pl.ds(0, ZROWS)],
                                deg_sh.at[pl.ds(sid * ZROWS, ZROWS)], shsem)

        c_st.wait()
        c_nd.wait()

        bbase = sid * BPT

        @pl.when(cid == 0)
        def _():
            gi0 = nodes_v.at[pl.ds(bbase, BPT)]
            pltpu.async_copy(x_hbm.at[gi0, pl.ds(0, H)], xlo, xgsem1)
            pltpu.async_copy(x_hbm.at[gi0, pl.ds(H, H)], xhi, xgsem2)

        @plsc.parallel_loop(0, B // L, unroll=4)
        def sbuild(i):
            nv = nodes_v[pl.ds(i * L, L)]
            plsc.store_scatter(slot_v, [nv], iota16 + i * L)

        c_zl.wait()
        c_zh.wait()
        c_zd.wait()
        c_ed.wait()
        c_es.wait()

        plsc.subcore_barrier()

        @plsc.parallel_loop(0, NVEC, unroll=4, carry=jnp.int32(0))
        def cpass(i, cnt):
            dv = dstbuf[pl.ds(i * L, L)]
            sv = srcbuf[pl.ds(i * L, L)]
            slotv = plsc.load_gather(slot_v, [dv])
            m = slotv != TRASH
            plsc.store_compressed(csrc.at[pl.ds(cnt, L)], sv, mask=m)
            plsc.store_compressed(cslot.at[pl.ds(cnt, L)], slotv, mask=m)
            pc = plsc.all_reduce_population_count(m)
            return cnt + pc[0]
        count = cpass

        trash16 = jnp.full((L,), TRASH, jnp.int32)
        zero16i = jnp.zeros((L,), jnp.int32)
        for i in range(K // L):
            cslot[pl.ds(count + i * L, L)] = trash16
            csrc[pl.ds(count + i * L, L)] = zero16i

        nchunk = (count + (K - 1)) // K

        def issue_gathers(c, b):
            base = c * K
            gi = csrc.at[pl.ds(base, K)]
            pltpu.async_copy(x_hbm.at[gi, pl.ds(0, H)], glos[b], gsems[b])
            pltpu.async_copy(x_hbm.at[gi, pl.ds(H, H)], ghis[b], gsems[b])

        @pl.when(nchunk > 0)
        def _():
            issue_gathers(0, 0)

        def wait_scatters(c, b):
            si = cslot.at[pl.ds(c * K, K)]
            pltpu.make_async_copy(glos[b], acc_lo.at[si], slsem).wait()
            pltpu.make_async_copy(ghis[b], acc_hi.at[si], shsem).wait()
            pltpu.make_async_copy(onesbuf, deg_sh.at[si], sdsem).wait()

        def outer(jo, _):
            for b in range(2):
                c = jo * 2 + b

                @pl.when(c < nchunk)
                def _():
                    base = c * K
                    gi = csrc.at[pl.ds(base, K)]
                    si = cslot.at[pl.ds(base, K)]
                    pltpu.make_async_copy(x_hbm.at[gi, pl.ds(0, H)], glos[b],
                                          gsems[b]).wait()
                    pltpu.make_async_copy(x_hbm.at[gi, pl.ds(H, H)], ghis[b],
                                          gsems[b]).wait()

                    @pl.when(c >= 1)
                    def _():
                        wait_scatters(c - 1, 1 - b)

                    @pl.when(c + 1 < nchunk)
                    def _():
                        issue_gathers(c + 1, 1 - b)

                    pltpu.async_copy(glos[b], acc_lo.at[si], slsem, add=True)
                    pltpu.async_copy(ghis[b], acc_hi.at[si], shsem, add=True)
                    pltpu.async_copy(onesbuf, deg_sh.at[si], sdsem, add=True)
            return 0
        lax.fori_loop(0, (nchunk + 1) // 2, outer, 0)

        @pl.when(nchunk > 0)
        def _():
            last = nchunk - 1

            @pl.when(last % 2 == 0)
            def _():
                wait_scatters(last, 0)

            @pl.when(last % 2 == 1)
            def _():
                wait_scatters(last, 1)

        obase = cid * B + bbase
        for i in range(BPT // L):
            nv = nodes_v[pl.ds(bbase + i * L, L)]
            slotv = plsc.load_gather(slot_v, [nv])
            eidx[pl.ds(i * L, L)] = slotv
            oidx[pl.ds(i * L, L)] = iota16 + (obase + i * L)

        @pl.when(cid == 0)
        def _():
            pltpu.make_async_copy(x_hbm.at[nodes_v.at[pl.ds(bbase, BPT)],
                                           pl.ds(0, H)], xlo, xgsem1).wait()
            pltpu.async_copy(xlo, xb_hbm.at[pl.ds(bbase, BPT), pl.ds(0, H)],
                             xgsem1)
            pltpu.make_async_copy(x_hbm.at[nodes_v.at[pl.ds(bbase, BPT)],
                                           pl.ds(H, H)], xhi, xgsem2).wait()
            pltpu.async_copy(xhi, xb_hbm.at[pl.ds(bbase, BPT), pl.ds(H, H)],
                             xgsem2)

        plsc.subcore_barrier()

        e1 = pltpu.async_copy(acc_lo.at[eidx], glo0.at[pl.ds(0, BPT)], gsem0)
        e2 = pltpu.async_copy(acc_hi.at[eidx], ghi0.at[pl.ds(0, BPT)], gsem1)
        e3 = pltpu.async_copy(deg_sh.at[eidx], degbuf.at[pl.ds(0, BPT)], sdsem)
        e1.wait()
        w1 = pltpu.async_copy(glo0.at[pl.ds(0, BPT)],
                              nlo_hbm.at[pl.ds(obase, BPT)], xsem1)
        e2.wait()
        w2 = pltpu.async_copy(ghi0.at[pl.ds(0, BPT)],
                              nhi_hbm.at[pl.ds(obase, BPT)], xsem2)
        e3.wait()
        w3 = pltpu.async_copy(degbuf.at[pl.ds(0, BPT)], dpart_hbm.at[oidx],
                              xsem3)

        @pl.when(cid == 0)
        def _():
            pltpu.make_async_copy(xlo, xb_hbm.at[pl.ds(bbase, BPT),
                                                 pl.ds(0, H)], xgsem1).wait()
            pltpu.make_async_copy(xhi, xb_hbm.at[pl.ds(bbase, BPT),
                                                 pl.ds(H, H)], xgsem2).wait()

        w1.wait()
        w2.wait()
        w3.wait()

    return sc(x, src, dst, nodes, zacc, trash_tab)


def _tc_body(nlo_ref, nhi_ref, dp_ref, xb_ref, wenc_ref, wcls_ref, out_ref):
    deg = dp_ref[0:B, 0:1] + dp_ref[B:2 * B, 0:1]
    inv = 1.0 / jnp.maximum(deg, 1.0)
    mean_lo = (nlo_ref[0:B, :] + nlo_ref[B:2 * B, :]) * inv
    mean_hi = (nhi_ref[0:B, :] + nhi_ref[B:2 * B, :]) * inv
    w1 = wenc_ref[:, 0:D]
    w2a = wenc_ref[:, D:D + H]
    w2b = wenc_ref[:, D + H:D + 2 * H]
    dn = (((1,), (1,)), ((), ()))
    mm = functools.partial(lax.dot_general, dimension_numbers=dn,
                           precision=lax.Precision.HIGHEST,
                           preferred_element_type=jnp.float32)
    h = mm(xb_ref[...], w1) + mm(mean_lo, w2a) + mm(mean_hi, w2b)
    h = jnp.maximum(h, 0.0)
    out_ref[...] = mm(h, wcls_ref[...])


def kernel(x, edge_index, nodes, W_enc, weight):
    src = edge_index[0]
    dst = edge_index[1]
    zacc = jnp.zeros((ZROWS, H), jnp.float32)
    trash_tab = jnp.full((SLOT_N,), TRASH, jnp.int32)
    nlo, nhi, dpart, xb = _sc_aggregate(x, src, dst, nodes, zacc, trash_tab)
    dpart = dpart.reshape(NC * B, 1)
    scores = pl.pallas_call(
        _tc_body,
        out_shape=jax.ShapeDtypeStruct((B, C), jnp.float32),
    )(nlo, nhi, dpart, xb, W_enc, weight)
    return scores


# speedup vs baseline: 1.1857x; 1.1857x over previous
import functools

import jax
import jax.numpy as jnp
from jax import lax
from jax.experimental import pallas as pl
from jax.experimental.pallas import tpu as pltpu
from jax.experimental.pallas import tpu_sc as plsc

N = 10000
E = 160000
D = 256
H = 128
EMBED = 256
C = 128
B = 1024

NC = 2
NS = 16
NW = NC * NS
L = 16

EPT = E // NW
EPT16 = 5008
NVEC = EPT16 // L
K = 128
CSIZE = EPT16 + K
SLOT_N = 10112
TRASH = B
ZROWS = 72
ACC_ROWS = NS * ZROWS
BPT = B // NS


def _sc_aggregate(x, src, dst, nodes, zacc, trash_tab):
    mesh = plsc.VectorSubcoreMesh(core_axis_name="c", subcore_axis_name="s",
                                  num_cores=NC, num_subcores=NS)

    @functools.partial(
        pl.kernel,
        out_type=(
            jax.ShapeDtypeStruct((NC * B, H), jnp.float32),
            jax.ShapeDtypeStruct((NC * B, H), jnp.float32),
            jax.ShapeDtypeStruct((NC * B,), jnp.float32),
            jax.ShapeDtypeStruct((B, D), jnp.float32),
        ),
        mesh=mesh,
        compiler_params=pltpu.CompilerParams(needs_layout_passes=False),
        scratch_types=[
            pltpu.VMEM((SLOT_N,), jnp.int32),
            pltpu.VMEM((B,), jnp.int32),
            pltpu.VMEM((EPT16,), jnp.int32),
            pltpu.VMEM((EPT16,), jnp.int32),
            pltpu.VMEM((CSIZE,), jnp.int32),
            pltpu.VMEM((CSIZE,), jnp.int32),
            pltpu.VMEM((K, H), jnp.float32),
            pltpu.VMEM((K, H), jnp.float32),
            pltpu.VMEM((K, H), jnp.float32),
            pltpu.VMEM((K, H), jnp.float32),
            pltpu.VMEM((BPT,), jnp.int32),
            pltpu.VMEM((BPT,), jnp.int32),
            pltpu.VMEM((BPT,), jnp.int32),
            pltpu.VMEM((K,), jnp.float32),
            pltpu.VMEM((K,), jnp.float32),
            pltpu.VMEM((80,), jnp.float32),
            pltpu.VMEM_SHARED((ACC_ROWS, H), jnp.float32),
            pltpu.VMEM_SHARED((ACC_ROWS, H), jnp.float32),
            pltpu.VMEM_SHARED((ACC_ROWS,), jnp.float32),
            pltpu.SemaphoreType.DMA,
            pltpu.SemaphoreType.DMA,
            pltpu.SemaphoreType.DMA,
            pltpu.SemaphoreType.DMA,
            pltpu.SemaphoreType.DMA,
            pltpu.SemaphoreType.DMA,
            pltpu.SemaphoreType.DMA,
            pltpu.SemaphoreType.DMA,
        ],
    )
    def sc(x_hbm, src_hbm, dst_hbm, nodes_hbm, zacc_hbm, trash_hbm,
           nlo_hbm, nhi_hbm, dpart_hbm, xb_hbm,
           slot_v, nodes_v, dstbuf, srcbuf, csrc, cslot,
           glo0, ghi0, glo1, ghi1,
           eidx, oidx, bidx, onesbuf, degbuf, zbuf,
           acc_lo, acc_hi, deg_sh,
           gsem0, gsem1, slsem, shsem, sdsem, xsem1, xsem2, xsem3):
        cid = lax.axis_index("c")
        sid = lax.axis_index("s")
        gwid = cid * NS + sid
        ebase = gwid * EPT

        iota16 = lax.iota(jnp.int32, L)
        glos = (glo0, glo1)
        ghis = (ghi0, ghi1)
        gsems = (gsem0, gsem1)

        c_zl = pltpu.async_copy(zacc_hbm, acc_lo.at[pl.ds(sid * ZROWS, ZROWS)],
                                xsem1)
        c_zh = pltpu.async_copy(zacc_hbm, acc_hi.at[pl.ds(sid * ZROWS, ZROWS)],
                                xsem2)
        c_st = pltpu.async_copy(trash_hbm, slot_v, xsem3)
        c_nd = pltpu.async_copy(nodes_hbm, nodes_v, gsem0)
        pad_dst = jnp.full((L,), N, jnp.int32)
        dstbuf[pl.ds(EPT16 - L, L)] = pad_dst
        c_ed = pltpu.async_copy(dst_hbm.at[pl.ds(ebase, EPT)],
                                dstbuf.at[pl.ds(0, EPT)], gsem1)
        c_es = pltpu.async_copy(src_hbm.at[pl.ds(ebase, EPT)],
                                srcbuf.at[pl.ds(0, EPT)], slsem)

        zero16 = jnp.zeros((L,), jnp.float32)
        for i in range(80 // L):
            zbuf[pl.ds(i * L, L)] = zero16
        one16 = jnp.ones((L,), jnp.float32)
        for i in range(K // L):
            onesbuf[pl.ds(i * L, L)] = one16
        c_zd = pltpu.async_copy(zbuf.at[pl.ds(0, ZROWS)],
                                deg_sh.at[pl.ds(sid * ZROWS, ZROWS)], shsem)

        c_st.wait()
        c_nd.wait()

        @plsc.parallel_loop(0, B // L, unroll=4)
        def sbuild(i):
            nv = nodes_v[pl.ds(i * L, L)]
            plsc.store_scatter(slot_v, [nv], iota16 + i * L)

        c_zl.wait()
        c_zh.wait()
        c_zd.wait()
        c_ed.wait()
        c_es.wait()

        plsc.subcore_barrier()

        @plsc.parallel_loop(0, NVEC, unroll=8, carry=jnp.int32(0))
        def cpass(i, cnt):
            dv = dstbuf[pl.ds(i * L, L)]
            sv = srcbuf[pl.ds(i * L, L)]
            slotv = plsc.load_gather(slot_v, [dv])
            m = slotv != TRASH
            plsc.store_compressed(csrc.at[pl.ds(cnt, L)], sv, mask=m)
            plsc.store_compressed(cslot.at[pl.ds(cnt, L)], slotv, mask=m)
            pc = plsc.all_reduce_population_count(m)
            return cnt + pc[0]
        count = cpass

        trash16 = jnp.full((L,), TRASH, jnp.int32)
        zero16i = jnp.zeros((L,), jnp.int32)
        for i in range(K // L):
            cslot[pl.ds(count + i * L, L)] = trash16
            csrc[pl.ds(count + i * L, L)] = zero16i

        nchunk = (count + (K - 1)) // K

        def issue_gathers(c, b):
            base = c * K
            gi = csrc.at[pl.ds(base, K)]
            pltpu.async_copy(x_hbm.at[gi, pl.ds(0, H)], glos[b], gsems[b])
            pltpu.async_copy(x_hbm.at[gi, pl.ds(H, H)], ghis[b], gsems[b])

        @pl.when(nchunk > 0)
        def _():
            issue_gathers(0, 0)

        def wait_scatters(c, b):
            si = cslot.at[pl.ds(c * K, K)]
            pltpu.make_async_copy(glos[b], acc_lo.at[si], slsem).wait()
            pltpu.make_async_copy(ghis[b], acc_hi.at[si], shsem).wait()
            pltpu.make_async_copy(onesbuf, deg_sh.at[si], sdsem).wait()

        def outer(jo, _):
            for b in range(2):
                c = jo * 2 + b

                @pl.when(c < nchunk)
                def _():
                    base = c * K
                    gi = csrc.at[pl.ds(base, K)]
                    si = cslot.at[pl.ds(base, K)]
                    pltpu.make_async_copy(x_hbm.at[gi, pl.ds(0, H)], glos[b],
                                          gsems[b]).wait()
                    pltpu.make_async_copy(x_hbm.at[gi, pl.ds(H, H)], ghis[b],
                                          gsems[b]).wait()

                    @pl.when(c >= 1)
                    def _():
                        wait_scatters(c - 1, 1 - b)

                    @pl.when(c + 1 < nchunk)
                    def _():
                        issue_gathers(c + 1, 1 - b)

                    pltpu.async_copy(glos[b], acc_lo.at[si], slsem, add=True)
                    pltpu.async_copy(ghis[b], acc_hi.at[si], shsem, add=True)
                    pltpu.async_copy(onesbuf, deg_sh.at[si], sdsem, add=True)
            return 0
        lax.fori_loop(0, (nchunk + 1) // 2, outer, 0)

        @pl.when(nchunk > 0)
        def _():
            last = nchunk - 1

            @pl.when(last % 2 == 0)
            def _():
                wait_scatters(last, 0)

            @pl.when(last % 2 == 1)
            def _():
                wait_scatters(last, 1)

        plsc.subcore_barrier()

        bbase = sid * BPT
        obase = cid * B + bbase
        for i in range(BPT // L):
            nv = nodes_v[pl.ds(bbase + i * L, L)]
            slotv = plsc.load_gather(slot_v, [nv])
            eidx[pl.ds(i * L, L)] = slotv
            bidx[pl.ds(i * L, L)] = nv
            oidx[pl.ds(i * L, L)] = iota16 + (obase + i * L)
        e1 = pltpu.async_copy(acc_lo.at[eidx], glo0.at[pl.ds(0, BPT)], gsem0)
        e2 = pltpu.async_copy(acc_hi.at[eidx], ghi0.at[pl.ds(0, BPT)], gsem1)
        e3 = pltpu.async_copy(deg_sh.at[eidx], degbuf.at[pl.ds(0, BPT)], sdsem)
        e1.wait()
        w1 = pltpu.async_copy(glo0.at[pl.ds(0, BPT)],
                              nlo_hbm.at[pl.ds(obase, BPT)], xsem1)
        e2.wait()
        w2 = pltpu.async_copy(ghi0.at[pl.ds(0, BPT)],
                              nhi_hbm.at[pl.ds(obase, BPT)], xsem2)
        e3.wait()
        w3 = pltpu.async_copy(degbuf.at[pl.ds(0, BPT)], dpart_hbm.at[oidx],
                              xsem3)

        @pl.when(cid == 0)
        def _():
            x1 = pltpu.async_copy(x_hbm.at[bidx, pl.ds(0, H)],
                                  glo1.at[pl.ds(0, BPT)], slsem)
            x2 = pltpu.async_copy(x_hbm.at[bidx, pl.ds(H, H)],
                                  ghi1.at[pl.ds(0, BPT)], shsem)
            x1.wait()
            w4 = pltpu.async_copy(glo1.at[pl.ds(0, BPT)],
                                  xb_hbm.at[pl.ds(bbase, BPT), pl.ds(0, H)],
                                  slsem)
            x2.wait()
            pltpu.sync_copy(ghi1.at[pl.ds(0, BPT)],
                            xb_hbm.at[pl.ds(bbase, BPT), pl.ds(H, H)])
            w4.wait()

        w1.wait()
        w2.wait()
        w3.wait()

    return sc(x, src, dst, nodes, zacc, trash_tab)


def _tc_body(nlo_ref, nhi_ref, dp_ref, xb_ref, wenc_ref, wcls_ref, out_ref):
    deg = dp_ref[0:B, 0:1] + dp_ref[B:2 * B, 0:1]
    inv = 1.0 / jnp.maximum(deg, 1.0)
    mean_lo = (nlo_ref[0:B, :] + nlo_ref[B:2 * B, :]) * inv
    mean_hi = (nhi_ref[0:B, :] + nhi_ref[B:2 * B, :]) * inv
    w1 = wenc_ref[:, 0:D]
    w2a = wenc_ref[:, D:D + H]
    w2b = wenc_ref[:, D + H:D + 2 * H]
    dn = (((1,), (1,)), ((), ()))
    mm = functools.partial(lax.dot_general, dimension_numbers=dn,
                           precision=lax.Precision.HIGHEST,
                           preferred_element_type=jnp.float32)
    h = mm(xb_ref[...], w1) + mm(mean_lo, w2a) + mm(mean_hi, w2b)
    h = jnp.maximum(h, 0.0)
    out_ref[...] = mm(h, wcls_ref[...])


def kernel(x, edge_index, nodes, W_enc, weight):
    src = edge_index[0]
    dst = edge_index[1]
    zacc = jnp.zeros((ZROWS, H), jnp.float32)
    trash_tab = jnp.full((SLOT_N,), TRASH, jnp.int32)
    nlo, nhi, dpart, xb = _sc_aggregate(x, src, dst, nodes, zacc, trash_tab)
    dpart = dpart.reshape(NC * B, 1)
    scores = pl.pallas_call(
        _tc_body,
        out_shape=jax.ShapeDtypeStruct((B, C), jnp.float32),
    )(nlo, nhi, dpart, xb, W_enc, weight)
    return scores


# speedup vs baseline: 1.1865x; 1.0007x over previous
import functools

import jax
import jax.numpy as jnp
from jax import lax
from jax.experimental import pallas as pl
from jax.experimental.pallas import tpu as pltpu
from jax.experimental.pallas import tpu_sc as plsc

N = 10000
E = 160000
D = 256
H = 128
EMBED = 256
C = 128
B = 1024

NC = 2
NS = 16
NW = NC * NS
L = 16

EPT = E // NW
EPT16 = 5008
NVEC = EPT16 // L
K = 128
CSIZE = EPT16 + K
SLOT_N = 10112
TRASH = B
ZROWS = 72
ACC_ROWS = NS * ZROWS
BPT = B // NS


def _sc_aggregate(x, src, dst, nodes, zacc, trash_tab):
    mesh = plsc.VectorSubcoreMesh(core_axis_name="c", subcore_axis_name="s",
                                  num_cores=NC, num_subcores=NS)

    @functools.partial(
        pl.kernel,
        out_type=(
            jax.ShapeDtypeStruct((NC * B, H), jnp.float32),
            jax.ShapeDtypeStruct((NC * B, H), jnp.float32),
            jax.ShapeDtypeStruct((NC * B,), jnp.float32),
            jax.ShapeDtypeStruct((B, D), jnp.float32),
        ),
        mesh=mesh,
        compiler_params=pltpu.CompilerParams(needs_layout_passes=False),
        scratch_types=[
            pltpu.VMEM((SLOT_N,), jnp.int32),
            pltpu.VMEM((B,), jnp.int32),
            pltpu.VMEM((EPT16,), jnp.int32),
            pltpu.VMEM((EPT16,), jnp.int32),
            pltpu.VMEM((CSIZE,), jnp.int32),
            pltpu.VMEM((CSIZE,), jnp.int32),
            pltpu.VMEM((K, H), jnp.float32),
            pltpu.VMEM((K, H), jnp.float32),
            pltpu.VMEM((K, H), jnp.float32),
            pltpu.VMEM((K, H), jnp.float32),
            pltpu.VMEM((BPT,), jnp.int32),
            pltpu.VMEM((BPT,), jnp.int32),
            pltpu.VMEM((BPT,), jnp.int32),
            pltpu.VMEM((K,), jnp.float32),
            pltpu.VMEM((K,), jnp.float32),
            pltpu.VMEM((80,), jnp.float32),
            pltpu.VMEM_SHARED((ACC_ROWS, H), jnp.float32),
            pltpu.VMEM_SHARED((ACC_ROWS, H), jnp.float32),
            pltpu.VMEM_SHARED((ACC_ROWS,), jnp.float32),
            pltpu.SemaphoreType.DMA,
            pltpu.SemaphoreType.DMA,
            pltpu.SemaphoreType.DMA,
            pltpu.SemaphoreType.DMA,
            pltpu.SemaphoreType.DMA,
            pltpu.SemaphoreType.DMA,
            pltpu.SemaphoreType.DMA,
            pltpu.SemaphoreType.DMA,
        ],
    )
    def sc(x_hbm, src_hbm, dst_hbm, nodes_hbm, zacc_hbm, trash_hbm,
           nlo_hbm, nhi_hbm, dpart_hbm, xb_hbm,
           slot_v, nodes_v, dstbuf, srcbuf, csrc, cslot,
           glo0, ghi0, glo1, ghi1,
           eidx, oidx, bidx, onesbuf, degbuf, zbuf,
           acc_lo, acc_hi, deg_sh,
           gsem0, gsem1, slsem, shsem, sdsem, xsem1, xsem2, xsem3):
        cid = lax.axis_index("c")
        sid = lax.axis_index("s")
        gwid = cid * NS + sid
        ebase = gwid * EPT

        iota16 = lax.iota(jnp.int32, L)
        glos = (glo0, glo1)
        ghis = (ghi0, ghi1)
        gsems = (gsem0, gsem1)

        c_zl = pltpu.async_copy(zacc_hbm, acc_lo.at[pl.ds(sid * ZROWS, ZROWS)],
                                xsem1)
        c_zh = pltpu.async_copy(zacc_hbm, acc_hi.at[pl.ds(sid * ZROWS, ZROWS)],
                                xsem2)
        c_st = pltpu.async_copy(trash_hbm, slot_v, xsem3)
        c_nd = pltpu.async_copy(nodes_hbm, nodes_v, gsem0)
        pad_dst = jnp.full((L,), N, jnp.int32)
        dstbuf[pl.ds(EPT16 - L, L)] = pad_dst
        c_ed = pltpu.async_copy(dst_hbm.at[pl.ds(ebase, EPT)],
                                dstbuf.at[pl.ds(0, EPT)], gsem1)
        c_es = pltpu.async_copy(src_hbm.at[pl.ds(ebase, EPT)],
                                srcbuf.at[pl.ds(0, EPT)], slsem)

        zero16 = jnp.zeros((L,), jnp.float32)
        for i in range(80 // L):
            zbuf[pl.ds(i * L, L)] = zero16
        one16 = jnp.ones((L,), jnp.float32)
        for i in range(K // L):
            onesbuf[pl.ds(i * L, L)] = one16
        c_zd = pltpu.async_copy(zbuf.at[pl.ds(0, ZROWS)],
                                deg_sh.at[pl.ds(sid * ZROWS, ZROWS)], shsem)

        c_st.wait()
        c_nd.wait()

        @plsc.parallel_loop(0, B // L, unroll=4)
        def sbuild(i):
            nv = nodes_v[pl.ds(i * L, L)]
            plsc.store_scatter(slot_v, [nv], iota16 + i * L)

        c_zl.wait()
        c_zh.wait()
        c_zd.wait()
        c_ed.wait()
        c_es.wait()

        plsc.subcore_barrier()

        @plsc.parallel_loop(0, NVEC, unroll=4, carry=jnp.int32(0))
        def cpass(i, cnt):
            dv = dstbuf[pl.ds(i * L, L)]
            sv = srcbuf[pl.ds(i * L, L)]
            slotv = plsc.load_gather(slot_v, [dv])
            m = slotv != TRASH
            plsc.store_compressed(csrc.at[pl.ds(cnt, L)], sv, mask=m)
            plsc.store_compressed(cslot.at[pl.ds(cnt, L)], slotv, mask=m)
            pc = plsc.all_reduce_population_count(m)
            return cnt + pc[0]
        count = cpass

        trash16 = jnp.full((L,), TRASH, jnp.int32)
        zero16i = jnp.zeros((L,), jnp.int32)
        for i in range(K // L):
            cslot[pl.ds(count + i * L, L)] = trash16
            csrc[pl.ds(count + i * L, L)] = zero16i

        nchunk = (count + (K - 1)) // K

        def issue_gathers(c, b):
            base = c * K
            gi = csrc.at[pl.ds(base, K)]
            pltpu.async_copy(x_hbm.at[gi, pl.ds(0, H)], glos[b], gsems[b])
            pltpu.async_copy(x_hbm.at[gi, pl.ds(H, H)], ghis[b], gsems[b])

        @pl.when(nchunk > 0)
        def _():
            issue_gathers(0, 0)

        def wait_scatters(c, b):
            si = cslot.at[pl.ds(c * K, K)]
            pltpu.make_async_copy(glos[b], acc_lo.at[si], slsem).wait()
            pltpu.make_async_copy(ghis[b], acc_hi.at[si], shsem).wait()
            pltpu.make_async_copy(onesbuf, deg_sh.at[si], sdsem).wait()

        def outer(jo, _):
            for b in range(2):
                c = jo * 2 + b

                @pl.when(c < nchunk)
                def _():
                    base = c * K
                    gi = csrc.at[pl.ds(base, K)]
                    si = cslot.at[pl.ds(base, K)]
                    pltpu.make_async_copy(x_hbm.at[gi, pl.ds(0, H)], glos[b],
                                          gsems[b]).wait()
                    pltpu.make_async_copy(x_hbm.at[gi, pl.ds(H, H)], ghis[b],
                                          gsems[b]).wait()

                    @pl.when(c >= 1)
                    def _():
                        wait_scatters(c - 1, 1 - b)

                    @pl.when(c + 1 < nchunk)
                    def _():
                        issue_gathers(c + 1, 1 - b)

                    pltpu.async_copy(glos[b], acc_lo.at[si], slsem, add=True)
                    pltpu.async_copy(ghis[b], acc_hi.at[si], shsem, add=True)
                    pltpu.async_copy(onesbuf, deg_sh.at[si], sdsem, add=True)
            return 0
        lax.fori_loop(0, (nchunk + 1) // 2, outer, 0)

        @pl.when(nchunk > 0)
        def _():
            last = nchunk - 1

            @pl.when(last % 2 == 0)
            def _():
                wait_scatters(last, 0)

            @pl.when(last % 2 == 1)
            def _():
                wait_scatters(last, 1)

        plsc.subcore_barrier()

        bbase = sid * BPT
        obase = cid * B + bbase
        for i in range(BPT // L):
            nv = nodes_v[pl.ds(bbase + i * L, L)]
            slotv = plsc.load_gather(slot_v, [nv])
            eidx[pl.ds(i * L, L)] = slotv
            bidx[pl.ds(i * L, L)] = nv
            oidx[pl.ds(i * L, L)] = iota16 + (obase + i * L)
        e1 = pltpu.async_copy(acc_lo.at[eidx], glo0.at[pl.ds(0, BPT)], gsem0)
        e2 = pltpu.async_copy(acc_hi.at[eidx], ghi0.at[pl.ds(0, BPT)], gsem1)
        e3 = pltpu.async_copy(deg_sh.at[eidx], degbuf.at[pl.ds(0, BPT)], sdsem)
        e1.wait()
        w1 = pltpu.async_copy(glo0.at[pl.ds(0, BPT)],
                              nlo_hbm.at[pl.ds(obase, BPT)], xsem1)
        e2.wait()
        w2 = pltpu.async_copy(ghi0.at[pl.ds(0, BPT)],
                              nhi_hbm.at[pl.ds(obase, BPT)], xsem2)
        e3.wait()
        w3 = pltpu.async_copy(degbuf.at[pl.ds(0, BPT)], dpart_hbm.at[oidx],
                              xsem3)

        @pl.when(cid == 0)
        def _():
            x1 = pltpu.async_copy(x_hbm.at[bidx, pl.ds(0, H)],
                                  glo1.at[pl.ds(0, BPT)], slsem)
            x2 = pltpu.async_copy(x_hbm.at[bidx, pl.ds(H, H)],
                                  ghi1.at[pl.ds(0, BPT)], shsem)
            x1.wait()
            w4 = pltpu.async_copy(glo1.at[pl.ds(0, BPT)],
                                  xb_hbm.at[pl.ds(bbase, BPT), pl.ds(0, H)],
                                  slsem)
            x2.wait()
            pltpu.sync_copy(ghi1.at[pl.ds(0, BPT)],
                            xb_hbm.at[pl.ds(bbase, BPT), pl.ds(H, H)])
            w4.wait()

        w1.wait()
        w2.wait()
        w3.wait()

    return sc(x, src, dst, nodes, zacc, trash_tab)


def _tc_body(nlo_ref, nhi_ref, dp_ref, xb_ref, wenc_ref, wcls_ref, out_ref):
    deg = dp_ref[0:B, 0:1] + dp_ref[B:2 * B, 0:1]
    inv = 1.0 / jnp.maximum(deg, 1.0)
    mean_lo = (nlo_ref[0:B, :] + nlo_ref[B:2 * B, :]) * inv
    mean_hi = (nhi_ref[0:B, :] + nhi_ref[B:2 * B, :]) * inv
    w1 = wenc_ref[:, 0:D]
    w2a = wenc_ref[:, D:D + H]
    w2b = wenc_ref[:, D + H:D + 2 * H]
    dn = (((1,), (1,)), ((), ()))
    mm = functools.partial(lax.dot_general, dimension_numbers=dn,
                           precision=lax.Precision.HIGHEST,
                           preferred_element_type=jnp.float32)
    h = mm(xb_ref[...], w1) + mm(mean_lo, w2a) + mm(mean_hi, w2b)
    h = jnp.maximum(h, 0.0)
    out_ref[...] = mm(h, wcls_ref[...])


def kernel(x, edge_index, nodes, W_enc, weight):
    src = edge_index[0]
    dst = edge_index[1]
    zacc = jnp.zeros((ZROWS, H), jnp.float32)
    trash_tab = jnp.full((SLOT_N,), TRASH, jnp.int32)
    nlo, nhi, dpart, xb = _sc_aggregate(x, src, dst, nodes, zacc, trash_tab)
    dpart = dpart.reshape(NC * B, 1)
    scores = pl.pallas_call(
        _tc_body,
        out_shape=jax.ShapeDtypeStruct((B, C), jnp.float32),
    )(nlo, nhi, dpart, xb, W_enc, weight)
    return scores
